# Initial kernel scaffold; baseline (speedup 1.0000x reference)
import jax
import jax.numpy as jnp
from jax.experimental import pallas as pl


def kernel(queries, keys, values, attn_mask):
    raise NotImplementedError("write your pallas kernel here")


# speedup vs baseline: 6.9166x; 6.9166x over previous
from math import ceil, sqrt

import jax
import jax.numpy as jnp
import numpy as np
from jax.experimental import pallas as pl
from jax.experimental.pallas import tpu as pltpu

_B, _L, _H, _D = 2, 2048, 12, 64
_E = _D // 2 + 1
_U = min(5 * ceil(np.log(_L)), _L)
_NEG = -1e30
_KT = 512


def _build_consts():
    kk = np.arange(_E, dtype=np.float64)[:, None]
    nn = np.arange(_D, dtype=np.float64)[None, :]
    w = np.where((np.arange(_E) == 0) | (np.arange(_E) == _D // 2), 1.0, 2.0)
    ang = 2.0 * np.pi * kk * nn / _D
    C = np.zeros((_D, _D))
    S2 = np.zeros((_D, _D))
    C[:_E] = (w[:, None] * np.cos(ang)) / _D
    S2[:_E] = (-w[:, None] * np.sin(ang)) / _D
    idx = np.asarray(jax.random.randint(jax.random.key(42), (_L, _U), 0, _L))
    P = np.zeros((_L, _L), dtype=np.int8)
    np.add.at(P, (np.repeat(np.arange(_L), _U), idx.ravel()), 1)
    return C.astype(np.float32), S2.astype(np.float32), P


_C, _S2, _P = _build_consts()


def _m_kernel(q_ref, kr_ref, ki_ref, p_ref, m_ref):
    q = q_ref[0, :, :]
    maxr = jnp.full((_L, 1), _NEG, jnp.float32)
    maxi = jnp.full((_L, 1), _NEG, jnp.float32)
    sums = jnp.zeros((_L, 1), jnp.float32)
    nt = ((1,), (1,)), ((), ())
    hi = jax.lax.Precision.HIGHEST
    for t in range(_L // _KT):
        krt = kr_ref[0, t * _KT:(t + 1) * _KT, :]
        kit = ki_ref[0, t * _KT:(t + 1) * _KT, :]
        pt = p_ref[:, t * _KT:(t + 1) * _KT]
        sr = jax.lax.dot_general(q, krt, nt, preferred_element_type=jnp.float32, precision=hi)
        si = jax.lax.dot_general(q, kit, nt, preferred_element_type=jnp.float32, precision=hi)
        pf = pt.astype(jnp.float32)
        sel = pf > 0.0
        maxr = jnp.maximum(maxr, jnp.max(jnp.where(sel, sr, _NEG), axis=1, keepdims=True))
        maxi = jnp.maximum(maxi, jnp.max(jnp.where(sel, si, _NEG), axis=1, keepdims=True))
        sums = sums + jnp.sum((sr + si) * pf, axis=1, keepdims=True)
    m_ref[0, :, :] = maxr + maxi - sums * (1.0 / _L)


def _ctx_kernel(v_ref, ur_ref, ui_ref, idx_ref, c_ref, s2_ref, o_ref):
    v = v_ref[0, :, :]
    hi = jax.lax.Precision.HIGHEST
    rows = (jnp.dot(ur_ref[0], c_ref[...], preferred_element_type=jnp.float32, precision=hi)
            + jnp.dot(ui_ref[0], s2_ref[...], preferred_element_type=jnp.float32, precision=hi))
    mean_v = jnp.mean(v, axis=0, keepdims=True)
    o_ref[0, :, :] = jnp.broadcast_to(mean_v, (_L, _D))
    for i in range(_U):
        r = idx_ref[0, 0, i]
        o_ref[0, pl.ds(r, 1), :] = rows[i:i + 1, :]


@jax.jit
def kernel(queries, keys, values, attn_mask):
    del attn_mask
    B, L, H, D = queries.shape
    BH = B * H
    P = jnp.asarray(_P)
    C = jnp.asarray(_C)
    S2 = jnp.asarray(_S2)

    q = jnp.transpose(queries, (0, 2, 1, 3))
    k = jnp.transpose(keys, (0, 2, 1, 3))
    v = jnp.transpose(values, (0, 2, 1, 3))
    q_ft = jnp.fft.rfft(q, axis=-1)
    k_ft = jnp.fft.rfft(k, axis=-1)
    v_ft = jnp.fft.rfft(v, axis=-1)

    pad = jnp.zeros((B, H, _L, 64 - _E), jnp.float32)
    q128 = jnp.concatenate([q_ft.real, pad, q_ft.imag, pad], axis=-1).reshape(BH, _L, 128)
    kr128 = jnp.concatenate([k_ft.real, pad, -k_ft.imag, pad], axis=-1).reshape(BH, _L, 128)
    ki128 = jnp.concatenate([k_ft.imag, pad, k_ft.real, pad], axis=-1).reshape(BH, _L, 128)

    fspec = pl.BlockSpec((1, _L, 128), lambda i: (i, 0, 0))
    m = pl.pallas_call(
        _m_kernel,
        grid=(BH,),
        in_specs=[fspec, fspec, fspec, pl.BlockSpec((_L, _L), lambda i: (0, 0))],
        out_specs=pl.BlockSpec((1, _L, 1), lambda i: (i, 0, 0)),
        out_shape=jax.ShapeDtypeStruct((BH, _L, 1), jnp.float32),
    )(q128, kr128, ki128, P)

    M_top = jax.lax.top_k(m[:, :, 0].reshape(B, H, _L), _U)[1]

    b_idx = jnp.arange(B)[:, None, None]
    h_idx = jnp.arange(H)[None, :, None]
    Q_reduce = q_ft[b_idx, h_idx, M_top, :]
    scores = jnp.matmul(Q_reduce, jnp.swapaxes(k_ft, -2, -1)) * (1.0 / sqrt(D))
    attn_r = jax.nn.softmax(scores.real, axis=-1)
    attn_i = jax.nn.softmax(scores.imag, axis=-1)
    upd_r = jnp.matmul(attn_r, v_ft.real)
    upd_i = jnp.matmul(attn_i, v_ft.imag)

    padu = jnp.zeros((B, H, _U, 64 - _E), jnp.float32)
    ur = jnp.concatenate([upd_r, padu], axis=-1).reshape(BH, _U, 64)
    ui = jnp.concatenate([upd_i, padu], axis=-1).reshape(BH, _U, 64)
    top3 = M_top.astype(jnp.int32).reshape(BH, 1, _U)
    vt = v.reshape(BH, _L, _D)

    const64 = pl.BlockSpec((_D, _D), lambda i: (0, 0))
    out = pl.pallas_call(
        _ctx_kernel,
        grid=(BH,),
        in_specs=[
            pl.BlockSpec((1, _L, _D), lambda i: (i, 0, 0)),
            pl.BlockSpec((1, _U, _D), lambda i: (i, 0, 0)),
            pl.BlockSpec((1, _U, _D), lambda i: (i, 0, 0)),
            pl.BlockSpec((1, 1, _U), lambda i: (i, 0, 0), memory_space=pltpu.SMEM),
            const64,
            const64,
        ],
        out_specs=pl.BlockSpec((1, _L, _D), lambda i: (i, 0, 0)),
        out_shape=jax.ShapeDtypeStruct((BH, _L, _D), jnp.float32),
    )(vt, ur, ui, top3, C, S2)
    return out.reshape(B, H, _L, _D)


# speedup vs baseline: 6.9201x; 1.0005x over previous
from math import ceil, sqrt

import jax
import jax.numpy as jnp
import numpy as np
from jax.experimental import pallas as pl
from jax.experimental.pallas import tpu as pltpu

_B, _L, _H, _D = 2, 2048, 12, 64
_E = _D // 2 + 1
_U = min(5 * ceil(np.log(_L)), _L)
_NEG = -1e30
_KT = 512


def _build_consts():
    kk = np.arange(_E, dtype=np.float64)[:, None]
    nn = np.arange(_D, dtype=np.float64)[None, :]
    w = np.where((np.arange(_E) == 0) | (np.arange(_E) == _D // 2), 1.0, 2.0)
    ang = 2.0 * np.pi * kk * nn / _D
    C = np.zeros((_D, _D))
    S2 = np.zeros((_D, _D))
    C[:_E] = (w[:, None] * np.cos(ang)) / _D
    S2[:_E] = (-w[:, None] * np.sin(ang)) / _D
    idx = np.asarray(jax.random.randint(jax.random.key(42), (_L, _U), 0, _L))
    P = np.zeros((_L, _L), dtype=np.int8)
    np.add.at(P, (np.repeat(np.arange(_L), _U), idx.ravel()), 1)
    return C.astype(np.float32), S2.astype(np.float32), P


_C, _S2, _P = _build_consts()


def _m_kernel(q_ref, kr_ref, ki_ref, p_ref, m_ref):
    q = q_ref[0, :, :]
    maxr = jnp.full((_L, 1), _NEG, jnp.float32)
    maxi = jnp.full((_L, 1), _NEG, jnp.float32)
    sums = jnp.zeros((_L, 1), jnp.float32)
    nt = ((1,), (1,)), ((), ())
    hi = jax.lax.Precision.HIGHEST
    for t in range(_L // _KT):
        krt = kr_ref[0, t * _KT:(t + 1) * _KT, :]
        kit = ki_ref[0, t * _KT:(t + 1) * _KT, :]
        pt = p_ref[:, t * _KT:(t + 1) * _KT]
        sr = jax.lax.dot_general(q, krt, nt, preferred_element_type=jnp.float32, precision=hi)
        si = jax.lax.dot_general(q, kit, nt, preferred_element_type=jnp.float32, precision=hi)
        pf = pt.astype(jnp.float32)
        sel = pf > 0.0
        maxr = jnp.maximum(maxr, jnp.max(jnp.where(sel, sr, _NEG), axis=1, keepdims=True))
        maxi = jnp.maximum(maxi, jnp.max(jnp.where(sel, si, _NEG), axis=1, keepdims=True))
        sums = sums + jnp.sum((sr + si) * pf, axis=1, keepdims=True)
    m_ref[0, :, :] = maxr + maxi - sums * (1.0 / _L)


def _ctx_kernel(v_ref, ur_ref, ui_ref, idx_ref, c_ref, s2_ref, o_ref):
    v = v_ref[0, :, :]
    hi = jax.lax.Precision.HIGHEST
    rows = (jnp.dot(ur_ref[0], c_ref[...], preferred_element_type=jnp.float32, precision=hi)
            + jnp.dot(ui_ref[0], s2_ref[...], preferred_element_type=jnp.float32, precision=hi))
    mean_v = jnp.mean(v, axis=0, keepdims=True)
    o_ref[0, :, :] = jnp.broadcast_to(mean_v, (_L, _D))
    for i in range(_U):
        r = idx_ref[0, 0, i]
        o_ref[0, pl.ds(r, 1), :] = rows[i:i + 1, :]


@jax.jit
def kernel(queries, keys, values, attn_mask):
    del attn_mask
    B, L, H, D = queries.shape
    BH = B * H
    P = jnp.asarray(_P)
    C = jnp.asarray(_C)
    S2 = jnp.asarray(_S2)

    q = jnp.transpose(queries, (0, 2, 1, 3))
    k = jnp.transpose(keys, (0, 2, 1, 3))
    v = jnp.transpose(values, (0, 2, 1, 3))
    q_ft = jnp.fft.rfft(q, axis=-1)
    k_ft = jnp.fft.rfft(k, axis=-1)
    v_ft = jnp.fft.rfft(v, axis=-1)

    pad = jnp.zeros((B, H, _L, 64 - _E), jnp.float32)
    q128 = jnp.concatenate([q_ft.real, pad, q_ft.imag, pad], axis=-1).reshape(BH, _L, 128)
    kr128 = jnp.concatenate([k_ft.real, pad, -k_ft.imag, pad], axis=-1).reshape(BH, _L, 128)
    ki128 = jnp.concatenate([k_ft.imag, pad, k_ft.real, pad], axis=-1).reshape(BH, _L, 128)

    fspec = pl.BlockSpec((1, _L, 128), lambda i: (i, 0, 0))
    m = pl.pallas_call(
        _m_kernel,
        grid=(BH,),
        in_specs=[fspec, fspec, fspec, pl.BlockSpec((_L, _L), lambda i: (0, 0))],
        out_specs=pl.BlockSpec((1, _L, 1), lambda i: (i, 0, 0)),
        out_shape=jax.ShapeDtypeStruct((BH, _L, 1), jnp.float32),
        compiler_params=pltpu.CompilerParams(dimension_semantics=("parallel",)),
    )(q128, kr128, ki128, P)

    M_top = jax.lax.top_k(m[:, :, 0].reshape(B, H, _L), _U)[1]

    b_idx = jnp.arange(B)[:, None, None]
    h_idx = jnp.arange(H)[None, :, None]
    Q_reduce = q_ft[b_idx, h_idx, M_top, :]
    scores = jnp.matmul(Q_reduce, jnp.swapaxes(k_ft, -2, -1)) * (1.0 / sqrt(D))
    attn_r = jax.nn.softmax(scores.real, axis=-1)
    attn_i = jax.nn.softmax(scores.imag, axis=-1)
    upd_r = jnp.matmul(attn_r, v_ft.real)
    upd_i = jnp.matmul(attn_i, v_ft.imag)

    padu = jnp.zeros((B, H, _U, 64 - _E), jnp.float32)
    ur = jnp.concatenate([upd_r, padu], axis=-1).reshape(BH, _U, 64)
    ui = jnp.concatenate([upd_i, padu], axis=-1).reshape(BH, _U, 64)
    top3 = M_top.astype(jnp.int32).reshape(BH, 1, _U)
    vt = v.reshape(BH, _L, _D)

    const64 = pl.BlockSpec((_D, _D), lambda i: (0, 0))
    out = pl.pallas_call(
        _ctx_kernel,
        grid=(BH,),
        in_specs=[
            pl.BlockSpec((1, _L, _D), lambda i: (i, 0, 0)),
            pl.BlockSpec((1, _U, _D), lambda i: (i, 0, 0)),
            pl.BlockSpec((1, _U, _D), lambda i: (i, 0, 0)),
            pl.BlockSpec((1, 1, _U), lambda i: (i, 0, 0), memory_space=pltpu.SMEM),
            const64,
            const64,
        ],
        out_specs=pl.BlockSpec((1, _L, _D), lambda i: (i, 0, 0)),
        out_shape=jax.ShapeDtypeStruct((BH, _L, _D), jnp.float32),
        compiler_params=pltpu.CompilerParams(dimension_semantics=("parallel",)),
    )(vt, ur, ui, top3, C, S2)
    return out.reshape(B, H, _L, _D)
